# baseline (speedup 1.0000x reference)
import functools

import jax
import jax.numpy as jnp
from jax.experimental import pallas as pl
from jax.experimental.pallas import tpu as pltpu

_N = 1024
_BJ = 16


def _gat_layer_kernel(cut_ref, wesc_ref, attsc_ref, s6_ref, x_ref, xT_ref,
                      ewT_ref, Wl_ref, bl_ref, Wr_ref, br_ref, attv_ref,
                      bias_ref, o_ref,
                      xlT_s, xlTrep_s, arow6_s, pj_s, pcm_s, acol_s, lg_s):
    d = attv_ref.shape[1]
    bj = x_ref.shape[0]
    n = ewT_ref.shape[1]
    r = d * bj
    nblk = n // bj
    pid = pl.program_id(0)

    @pl.when(pid == 0)
    def _init():
        attv = attv_ref[...]
        xlT = jnp.dot(Wl_ref[...], xT_ref[...],
                      preferred_element_type=jnp.float32) + bl_ref[...]
        xlT_s[...] = xlT
        arow6_s[...] = 0.6 * jnp.dot(attv, xlT, preferred_element_type=jnp.float32)
        pj_s[...] = (jax.lax.broadcasted_iota(jnp.int32, (r, bj), 0) % bj ==
                     jax.lax.broadcasted_iota(jnp.int32, (r, bj), 1)
                     ).astype(jnp.float32)
        pcm_s[...] = (jax.lax.broadcasted_iota(jnp.int32, (r, d), 0) // bj ==
                      jax.lax.broadcasted_iota(jnp.int32, (r, d), 1)
                      ).astype(jnp.float32)
        xlTrep_s[...] = jnp.dot(pcm_s[...], xlT,
                                preferred_element_type=jnp.float32
                                ).astype(jnp.bfloat16)

    @pl.when(pid < nblk)
    def _block():
        e = ewT_ref[...]
        x_b = x_ref[...]

        xr_b = jax.lax.dot_general(x_b, Wr_ref[...], (((1,), (1,)), ((), ())),
                                   preferred_element_type=jnp.float32) \
            + br_ref[...].reshape(1, d)
        b6 = 0.6 * jnp.sum(xr_b * attv_ref[...], axis=1, keepdims=True)
        xrrep = jnp.dot(pj_s[...], xr_b, preferred_element_type=jnp.float32)
        acol_s[...] = jnp.dot(xrrep * pcm_s[...], jnp.ones((d, 1), jnp.float32),
                              preferred_element_type=jnp.float32
                              ).astype(jnp.bfloat16)

        cchunk = 64

        def chan(c0, acc):
            base = c0 * cchunk
            upd = None
            for k in range(cchunk):
                c = base + k
                wec = wesc_ref[0, c].astype(jnp.bfloat16)
                attc4 = (attsc_ref[0, c] * 0.4).astype(jnp.bfloat16)
                colc = acol_s[pl.ds(pl.multiple_of(c * bj, bj), bj), :]
                rowc = xlTrep_s[pl.ds(pl.multiple_of(c * bj, bj), bj), :]
                m = e * wec.astype(jnp.float32) + rowc + colc
                t = jnp.abs(m) * attc4
                upd = t if upd is None else upd + t
            return acc + upd.astype(jnp.float32)

        T = jax.lax.fori_loop(0, d // cchunk, chan, jnp.zeros_like(e))
        logits = T + (e * s6_ref[0, 0] + arow6_s[...] + b6)
        logits = jnp.where(e > cut_ref[0, 0], logits, jnp.float32(-jnp.inf))
        lg_s[pl.ds(pl.multiple_of(pid * bj, bj), bj), :] = logits

    @pl.when(pid == nblk)
    def _softmax():
        neginf = jnp.float32(-jnp.inf)
        logits = lg_s[...]
        rmax = jnp.max(logits, axis=1, keepdims=True)
        cm = jnp.where(rmax > neginf, rmax, jnp.float32(0.0))
        ex = jnp.exp(logits - cm)
        den = jnp.sum(ex, axis=1, keepdims=True)
        alpha = ex / (den + jnp.float32(1e-16))
        out = jax.lax.dot_general(alpha, xlT_s[...], (((1,), (1,)), ((), ())),
                                  preferred_element_type=jnp.float32)
        o_ref[...] = jnp.maximum(out + bias_ref[...], jnp.float32(0.0))


def _gat_layer(x, xT, ewT, cutoff, Wl, bl, Wr, br, We, att, bias, *, bj=_BJ):
    n, cin = x.shape
    d = Wl.shape[0]
    r = d * bj
    nblk = n // bj
    grid = (nblk + 1,)
    smem = pl.BlockSpec(memory_space=pltpu.SMEM)
    s6 = (0.6 * jnp.sum(att * We.reshape(-1))).reshape(1, 1)
    clamp = lambda j: (jnp.minimum(j, nblk - 1), 0)
    return pl.pallas_call(
        _gat_layer_kernel,
        grid=grid,
        in_specs=[
            smem,
            smem,
            smem,
            smem,
            pl.BlockSpec((bj, cin), clamp),
            pl.BlockSpec((cin, n), lambda j: (0, 0)),
            pl.BlockSpec((bj, n), clamp),
            pl.BlockSpec((d, cin), lambda j: (0, 0)),
            pl.BlockSpec((d, 1), lambda j: (0, 0)),
            pl.BlockSpec((d, cin), lambda j: (0, 0)),
            pl.BlockSpec((d, 1), lambda j: (0, 0)),
            pl.BlockSpec((1, d), lambda j: (0, 0)),
            pl.BlockSpec((1, d), lambda j: (0, 0)),
        ],
        out_specs=pl.BlockSpec((n, d), lambda j: (0, 0)),
        out_shape=jax.ShapeDtypeStruct((n, d), jnp.float32),
        scratch_shapes=[
            pltpu.VMEM((d, n), jnp.float32),
            pltpu.VMEM((r, n), jnp.bfloat16),
            pltpu.VMEM((1, n), jnp.float32),
            pltpu.VMEM((r, bj), jnp.float32),
            pltpu.VMEM((r, d), jnp.float32),
            pltpu.VMEM((r, 1), jnp.bfloat16),
            pltpu.VMEM((n, n), jnp.float32),
        ],
    )(cutoff, We.reshape(1, d), att.reshape(1, d), s6, x, xT, ewT,
      Wl, bl.reshape(d, 1), Wr, br.reshape(d, 1),
      att.reshape(1, d), bias.reshape(1, d))


def _conv_mat(w_ref, lin, lout, stride):
    im = jax.lax.broadcasted_iota(jnp.int32, (lin, lout), 0)
    il = jax.lax.broadcasted_iota(jnp.int32, (lin, lout), 1)
    a = jnp.zeros((lin, lout), jnp.float32)
    for k in range(5):
        a = a + jnp.where(im == stride * il + 3 * k, w_ref[0, k], jnp.float32(0.0))
    return a


def _tail_kernel(x_ref, w1_ref, b1_ref, w2_ref, b2_ref, w3_ref, b3_ref,
                 lw_ref, lb_ref, o_ref):
    xm = jnp.sum(x_ref[...], axis=0, keepdims=True) * jnp.float32(1.0 / _N)
    y = jnp.maximum(jnp.dot(xm, _conv_mat(w1_ref, 64, 52, 1),
                            preferred_element_type=jnp.float32) + b1_ref[0, 0], 0.0)
    y = jnp.maximum(jnp.dot(y, _conv_mat(w2_ref, 52, 40, 1),
                            preferred_element_type=jnp.float32) + b2_ref[0, 0], 0.0)
    y = jnp.maximum(jnp.dot(y, _conv_mat(w3_ref, 40, 14, 2),
                            preferred_element_type=jnp.float32) + b3_ref[0, 0], 0.0)
    o = jnp.sum(y * lw_ref[...], axis=1, keepdims=True) + lb_ref[0, 0]
    o_ref[...] = o


def _tail(x, c1_w, c1_b, c2_w, c2_b, c3_w, c3_b, l1_W, l1_b):
    smem = pl.BlockSpec(memory_space=pltpu.SMEM)
    vmem = pl.BlockSpec(memory_space=pltpu.VMEM)
    return pl.pallas_call(
        _tail_kernel,
        in_specs=[vmem, smem, smem, smem, smem, smem, smem, vmem, smem],
        out_specs=vmem,
        out_shape=jax.ShapeDtypeStruct((1, 1), jnp.float32),
    )(x, c1_w.reshape(1, 5), c1_b.reshape(1, 1), c2_w.reshape(1, 5),
      c2_b.reshape(1, 1), c3_w.reshape(1, 5), c3_b.reshape(1, 1),
      l1_W, l1_b.reshape(1, 1))


def kernel(features, edge_weights, threashold,
           conv1_Wl, conv1_bl, conv1_Wr, conv1_br, conv1_We, conv1_att, conv1_bias,
           conv2_Wl, conv2_bl, conv2_Wr, conv2_br, conv2_We, conv2_att, conv2_bias,
           conv3_Wl, conv3_bl, conv3_Wr, conv3_br, conv3_We, conv3_att, conv3_bias,
           conv4_Wl, conv4_bl, conv4_Wr, conv4_br, conv4_We, conv4_att, conv4_bias,
           c1_w, c1_b, c2_w, c2_b, c3_w, c3_b, l1_W, l1_b):
    cutoff = (jnp.float32(1.0) / threashold).astype(jnp.float32).reshape(1, 1)
    ewT = edge_weights.T
    x = features
    layer_ws = [
        (conv1_Wl, conv1_bl, conv1_Wr, conv1_br, conv1_We, conv1_att, conv1_bias),
        (conv2_Wl, conv2_bl, conv2_Wr, conv2_br, conv2_We, conv2_att, conv2_bias),
        (conv3_Wl, conv3_bl, conv3_Wr, conv3_br, conv3_We, conv3_att, conv3_bias),
        (conv4_Wl, conv4_bl, conv4_Wr, conv4_br, conv4_We, conv4_att, conv4_bias),
    ]
    for (Wl, bl, Wr, br, We, att, bias) in layer_ws:
        x = _gat_layer(x, x.T, ewT, cutoff, Wl, bl, Wr, br, We, att, bias)
    return _tail(x, c1_w, c1_b, c2_w, c2_b, c3_w, c3_b, l1_W, l1_b)


# speedup vs baseline: 1.3420x; 1.3420x over previous
import functools

import jax
import jax.numpy as jnp
from jax.experimental import pallas as pl
from jax.experimental.pallas import tpu as pltpu

_N = 1024
_BJ = 16


def _gat_layer_kernel(cut_ref, wesc_ref, attsc_ref, s6_ref, x_ref, xT_ref,
                      ewT_ref, Wl_ref, bl_ref, Wr_ref, br_ref, attv_ref,
                      bias_ref, o_ref,
                      xlT_s, xlTrep_s, arow6_s, pj_s, pcm_s, acol_s, lg_s):
    d = attv_ref.shape[1]
    bj = x_ref.shape[0]
    n = ewT_ref.shape[1]
    r = d * bj
    nblk = n // bj
    pid = pl.program_id(0)

    @pl.when(pid == 0)
    def _init():
        attv = attv_ref[...]
        xlT = jnp.dot(Wl_ref[...], xT_ref[...],
                      preferred_element_type=jnp.float32) + bl_ref[...]
        xlT_s[...] = xlT
        arow6_s[...] = 0.6 * jnp.dot(attv, xlT, preferred_element_type=jnp.float32)
        pj_s[...] = (jax.lax.broadcasted_iota(jnp.int32, (r, bj), 0) % bj ==
                     jax.lax.broadcasted_iota(jnp.int32, (r, bj), 1)
                     ).astype(jnp.float32)
        pcm_s[...] = (jax.lax.broadcasted_iota(jnp.int32, (r, d), 0) // bj ==
                      jax.lax.broadcasted_iota(jnp.int32, (r, d), 1)
                      ).astype(jnp.float32)
        xlTrep_s[...] = jnp.dot(pcm_s[...], xlT,
                                preferred_element_type=jnp.float32
                                ).astype(jnp.bfloat16)

    @pl.when(pid < nblk)
    def _block():
        e = ewT_ref[...]
        x_b = x_ref[...]

        xr_b = jax.lax.dot_general(x_b, Wr_ref[...], (((1,), (1,)), ((), ())),
                                   preferred_element_type=jnp.float32) \
            + br_ref[...].reshape(1, d)
        b6 = 0.6 * jnp.sum(xr_b * attv_ref[...], axis=1, keepdims=True)
        xrrep = jnp.dot(pj_s[...], xr_b, preferred_element_type=jnp.float32)
        acol_s[...] = jnp.dot(xrrep * pcm_s[...], jnp.ones((d, 1), jnp.float32),
                              preferred_element_type=jnp.float32
                              ).astype(jnp.bfloat16)
        ebf = e.astype(jnp.bfloat16)

        cchunk = 64

        def chan(c0, acc):
            base = c0 * cchunk
            upds = [None, None]
            for k in range(cchunk):
                c = base + k
                wec = wesc_ref[0, c].astype(jnp.bfloat16)
                attc4 = (attsc_ref[0, c] * 0.4).astype(jnp.bfloat16)
                colc = acol_s[pl.ds(pl.multiple_of(c * bj, bj), bj), :]
                rowc = xlTrep_s[pl.ds(pl.multiple_of(c * bj, bj), bj), :]
                m = ebf * wec + rowc + colc
                t = jnp.abs(m) * attc4
                upds[k % 2] = t if upds[k % 2] is None else upds[k % 2] + t
            return acc + (upds[0].astype(jnp.float32) + upds[1].astype(jnp.float32))

        T = jax.lax.fori_loop(0, d // cchunk, chan, jnp.zeros_like(e))
        logits = T + (e * s6_ref[0, 0] + arow6_s[...] + b6)
        logits = jnp.where(e > cut_ref[0, 0], logits, jnp.float32(-jnp.inf))
        lg_s[pl.ds(pl.multiple_of(pid * bj, bj), bj), :] = logits

    @pl.when(pid == nblk)
    def _softmax():
        neginf = jnp.float32(-jnp.inf)
        logits = lg_s[...]
        rmax = jnp.max(logits, axis=1, keepdims=True)
        cm = jnp.where(rmax > neginf, rmax, jnp.float32(0.0))
        ex = jnp.exp(logits - cm)
        den = jnp.sum(ex, axis=1, keepdims=True)
        alpha = ex / (den + jnp.float32(1e-16))
        out = jax.lax.dot_general(alpha, xlT_s[...], (((1,), (1,)), ((), ())),
                                  preferred_element_type=jnp.float32)
        o_ref[...] = jnp.maximum(out + bias_ref[...], jnp.float32(0.0))


def _gat_layer(x, xT, ewT, cutoff, Wl, bl, Wr, br, We, att, bias, *, bj=_BJ):
    n, cin = x.shape
    d = Wl.shape[0]
    r = d * bj
    nblk = n // bj
    grid = (nblk + 1,)
    smem = pl.BlockSpec(memory_space=pltpu.SMEM)
    s6 = (0.6 * jnp.sum(att * We.reshape(-1))).reshape(1, 1)
    clamp = lambda j: (jnp.minimum(j, nblk - 1), 0)
    return pl.pallas_call(
        _gat_layer_kernel,
        grid=grid,
        in_specs=[
            smem,
            smem,
            smem,
            smem,
            pl.BlockSpec((bj, cin), clamp),
            pl.BlockSpec((cin, n), lambda j: (0, 0)),
            pl.BlockSpec((bj, n), clamp),
            pl.BlockSpec((d, cin), lambda j: (0, 0)),
            pl.BlockSpec((d, 1), lambda j: (0, 0)),
            pl.BlockSpec((d, cin), lambda j: (0, 0)),
            pl.BlockSpec((d, 1), lambda j: (0, 0)),
            pl.BlockSpec((1, d), lambda j: (0, 0)),
            pl.BlockSpec((1, d), lambda j: (0, 0)),
        ],
        out_specs=pl.BlockSpec((n, d), lambda j: (0, 0)),
        out_shape=jax.ShapeDtypeStruct((n, d), jnp.float32),
        scratch_shapes=[
            pltpu.VMEM((d, n), jnp.float32),
            pltpu.VMEM((r, n), jnp.bfloat16),
            pltpu.VMEM((1, n), jnp.float32),
            pltpu.VMEM((r, bj), jnp.float32),
            pltpu.VMEM((r, d), jnp.float32),
            pltpu.VMEM((r, 1), jnp.bfloat16),
            pltpu.VMEM((n, n), jnp.float32),
        ],
    )(cutoff, We.reshape(1, d), att.reshape(1, d), s6, x, xT, ewT,
      Wl, bl.reshape(d, 1), Wr, br.reshape(d, 1),
      att.reshape(1, d), bias.reshape(1, d))


def _conv_mat(w_ref, lin, lout, stride):
    im = jax.lax.broadcasted_iota(jnp.int32, (lin, lout), 0)
    il = jax.lax.broadcasted_iota(jnp.int32, (lin, lout), 1)
    a = jnp.zeros((lin, lout), jnp.float32)
    for k in range(5):
        a = a + jnp.where(im == stride * il + 3 * k, w_ref[0, k], jnp.float32(0.0))
    return a


def _tail_kernel(x_ref, w1_ref, b1_ref, w2_ref, b2_ref, w3_ref, b3_ref,
                 lw_ref, lb_ref, o_ref):
    xm = jnp.sum(x_ref[...], axis=0, keepdims=True) * jnp.float32(1.0 / _N)
    y = jnp.maximum(jnp.dot(xm, _conv_mat(w1_ref, 64, 52, 1),
                            preferred_element_type=jnp.float32) + b1_ref[0, 0], 0.0)
    y = jnp.maximum(jnp.dot(y, _conv_mat(w2_ref, 52, 40, 1),
                            preferred_element_type=jnp.float32) + b2_ref[0, 0], 0.0)
    y = jnp.maximum(jnp.dot(y, _conv_mat(w3_ref, 40, 14, 2),
                            preferred_element_type=jnp.float32) + b3_ref[0, 0], 0.0)
    o = jnp.sum(y * lw_ref[...], axis=1, keepdims=True) + lb_ref[0, 0]
    o_ref[...] = o


def _tail(x, c1_w, c1_b, c2_w, c2_b, c3_w, c3_b, l1_W, l1_b):
    smem = pl.BlockSpec(memory_space=pltpu.SMEM)
    vmem = pl.BlockSpec(memory_space=pltpu.VMEM)
    return pl.pallas_call(
        _tail_kernel,
        in_specs=[vmem, smem, smem, smem, smem, smem, smem, vmem, smem],
        out_specs=vmem,
        out_shape=jax.ShapeDtypeStruct((1, 1), jnp.float32),
    )(x, c1_w.reshape(1, 5), c1_b.reshape(1, 1), c2_w.reshape(1, 5),
      c2_b.reshape(1, 1), c3_w.reshape(1, 5), c3_b.reshape(1, 1),
      l1_W, l1_b.reshape(1, 1))


def kernel(features, edge_weights, threashold,
           conv1_Wl, conv1_bl, conv1_Wr, conv1_br, conv1_We, conv1_att, conv1_bias,
           conv2_Wl, conv2_bl, conv2_Wr, conv2_br, conv2_We, conv2_att, conv2_bias,
           conv3_Wl, conv3_bl, conv3_Wr, conv3_br, conv3_We, conv3_att, conv3_bias,
           conv4_Wl, conv4_bl, conv4_Wr, conv4_br, conv4_We, conv4_att, conv4_bias,
           c1_w, c1_b, c2_w, c2_b, c3_w, c3_b, l1_W, l1_b):
    cutoff = (jnp.float32(1.0) / threashold).astype(jnp.float32).reshape(1, 1)
    ewT = edge_weights.T
    x = features
    layer_ws = [
        (conv1_Wl, conv1_bl, conv1_Wr, conv1_br, conv1_We, conv1_att, conv1_bias),
        (conv2_Wl, conv2_bl, conv2_Wr, conv2_br, conv2_We, conv2_att, conv2_bias),
        (conv3_Wl, conv3_bl, conv3_Wr, conv3_br, conv3_We, conv3_att, conv3_bias),
        (conv4_Wl, conv4_bl, conv4_Wr, conv4_br, conv4_We, conv4_att, conv4_bias),
    ]
    for (Wl, bl, Wr, br, We, att, bias) in layer_ws:
        x = _gat_layer(x, x.T, ewT, cutoff, Wl, bl, Wr, br, We, att, bias)
    return _tail(x, c1_w, c1_b, c2_w, c2_b, c3_w, c3_b, l1_W, l1_b)
